# Initial kernel scaffold; baseline (speedup 1.0000x reference)
import jax
import jax.numpy as jnp
from jax.experimental import pallas as pl


def kernel(input_ids, token_type_ids, word_table, pos_table, type_table, name_table, gamma, beta):
    raise NotImplementedError("write your pallas kernel here")


# speedup vs baseline: 1.0259x; 1.0259x over previous
import functools

import jax
import jax.numpy as jnp
import numpy as np
from jax import lax
from jax.experimental import pallas as pl
from jax.experimental.pallas import tpu as pltpu
from jax.experimental.pallas import tpu_sc as plsc

B = 32
S = 512
HIDDEN = 768
LANES = 16
NH = HIDDEN // LANES
CHUNK = 64
NCHUNK = S // CHUNK
EPS = 1e-12
N_TOK = B * S


def _hsum(v):
    lanes = lax.iota(jnp.int32, LANES)
    for k in (1, 2, 4, 8):
        rot = lax.bitwise_and(lanes + k, LANES - 1)
        v = v + v.at[rot].get(mode="promise_in_bounds")
    return v


def _sc_body(ids_hbm, idx2_hbm, word_hbm, comb_hbm, gamma_hbm, beta_hbm,
             out_hbm, ids_v, idx2_v, wbuf, cbuf, gv, bv, sem_w, sem_c):
    wid = lax.axis_index("s") * 2 + lax.axis_index("c")
    row_base = pl.multiple_of(wid * S, S)

    pltpu.sync_copy(ids_hbm.at[pl.ds(row_base, S)], ids_v)
    pltpu.sync_copy(idx2_hbm.at[pl.ds(row_base, S)], idx2_v)
    pltpu.sync_copy(gamma_hbm, gv)
    pltpu.sync_copy(beta_hbm, bv)

    def chunk_body(c, carry):
        base = pl.multiple_of(c * CHUNK, CHUNK)
        cp_w = pltpu.async_copy(
            word_hbm.at[ids_v.at[pl.ds(base, CHUNK)]], wbuf, sem_w)
        cp_c = pltpu.async_copy(
            comb_hbm.at[idx2_v.at[pl.ds(base, CHUNK)]], cbuf, sem_c)
        cp_w.wait()
        cp_c.wait()

        def tok_body(t, tc):
            acc_s = jnp.zeros((LANES,), jnp.float32)
            acc_q = jnp.zeros((LANES,), jnp.float32)
            for h in range(NH):
                sl = pl.ds(h * LANES, LANES)
                v = wbuf[t, sl] + cbuf[t, sl]
                wbuf[t, sl] = v
                acc_s = acc_s + v
                acc_q = acc_q + v * v
            mean_v = _hsum(acc_s) * (1.0 / HIDDEN)
            var_v = _hsum(acc_q) * (1.0 / HIDDEN) - mean_v * mean_v
            x = var_v + EPS
            bits = lax.bitcast_convert_type(x, jnp.int32)
            bits = jnp.int32(0x5F3759DF) - lax.shift_right_logical(
                bits, jnp.int32(1))
            y = lax.bitcast_convert_type(bits, jnp.float32)
            for _ in range(4):
                y = y * (1.5 - 0.5 * x * y * y)
            for h in range(NH):
                sl = pl.ds(h * LANES, LANES)
                v = wbuf[t, sl]
                wbuf[t, sl] = (v - mean_v) * y * gv[sl] + bv[sl]
            return tc

        lax.fori_loop(0, CHUNK, tok_body, 0)
        pltpu.sync_copy(wbuf, out_hbm.at[pl.ds(row_base + base, CHUNK)])
        return carry

    lax.fori_loop(0, NCHUNK, chunk_body, 0)


_sc_embed = functools.partial(
    pl.kernel,
    mesh=plsc.VectorSubcoreMesh(core_axis_name="c", subcore_axis_name="s"),
    out_type=jax.ShapeDtypeStruct((N_TOK, HIDDEN), jnp.float32),
    scratch_types=[
        pltpu.VMEM((S,), jnp.int32),
        pltpu.VMEM((S,), jnp.int32),
        pltpu.VMEM((CHUNK, HIDDEN), jnp.float32),
        pltpu.VMEM((CHUNK, HIDDEN), jnp.float32),
        pltpu.VMEM((HIDDEN,), jnp.float32),
        pltpu.VMEM((HIDDEN,), jnp.float32),
        pltpu.SemaphoreType.DMA,
        pltpu.SemaphoreType.DMA,
    ],
)(_sc_body)


def kernel(input_ids, token_type_ids, word_table, pos_table, type_table,
           name_table, gamma, beta):
    ids = input_ids.astype(jnp.int32).reshape(N_TOK)
    comb = (pos_table[None, :, :] + type_table[:, None, :]
            + name_table[0][None, None, :]).reshape(2 * S, HIDDEN)
    idx2 = (token_type_ids.astype(jnp.int32) * S
            + jnp.arange(S, dtype=jnp.int32)[None, :]).reshape(N_TOK)
    out = _sc_embed(ids, idx2, word_table, comb, gamma, beta)
    return out.reshape(B, S, HIDDEN)


# speedup vs baseline: 2.3117x; 2.2533x over previous
import functools

import jax
import jax.numpy as jnp
from jax import lax
from jax.experimental import pallas as pl
from jax.experimental.pallas import tpu as pltpu
from jax.experimental.pallas import tpu_sc as plsc

B = 32
S = 512
HIDDEN = 768
EPS = 1e-12
N_TOK = B * S
CHUNK = 64
NCHUNK = S // CHUNK
TBLK = 256


def _sc_gather_body(ids_hbm, word_hbm, out_hbm, ids_v, buf0, buf1,
                    sem_g0, sem_g1, sem_o0, sem_o1):
    wid = lax.axis_index("s") * 2 + lax.axis_index("c")
    row_base = pl.multiple_of(wid * S, S)
    pltpu.sync_copy(ids_hbm.at[pl.ds(row_base, S)], ids_v)

    bufs = (buf0, buf1)
    gsems = (sem_g0, sem_g1)
    osems = (sem_o0, sem_o1)

    def start_gather(c):
        p = c % 2
        return pltpu.async_copy(
            word_hbm.at[ids_v.at[pl.ds(c * CHUNK, CHUNK)]], bufs[p], gsems[p])

    def start_out(c):
        p = c % 2
        return pltpu.async_copy(
            bufs[p], out_hbm.at[pl.ds(row_base + c * CHUNK, CHUNK)], osems[p])

    gather_h = {0: start_gather(0)}
    out_h = {}
    for c in range(NCHUNK):
        if c + 1 < NCHUNK:
            if c - 1 >= 0:
                out_h[c - 1].wait()
            gather_h[c + 1] = start_gather(c + 1)
        gather_h[c].wait()
        out_h[c] = start_out(c)
    out_h[NCHUNK - 2].wait()
    out_h[NCHUNK - 1].wait()


_sc_gather = functools.partial(
    pl.kernel,
    mesh=plsc.VectorSubcoreMesh(core_axis_name="c", subcore_axis_name="s"),
    out_type=jax.ShapeDtypeStruct((N_TOK, HIDDEN), jnp.float32),
    scratch_types=[
        pltpu.VMEM((S,), jnp.int32),
        pltpu.VMEM((CHUNK, HIDDEN), jnp.float32),
        pltpu.VMEM((CHUNK, HIDDEN), jnp.float32),
        pltpu.SemaphoreType.DMA,
        pltpu.SemaphoreType.DMA,
        pltpu.SemaphoreType.DMA,
        pltpu.SemaphoreType.DMA,
    ],
)(_sc_gather_body)


def _tc_ln_body(gath_ref, pp_ref, tf_ref, diff_ref, g_ref, b_ref, o_ref):
    x = (gath_ref[0] + pp_ref[...]
         + tf_ref[0, 0, :][:, None] * diff_ref[0][None, :])
    mean = jnp.mean(x, axis=-1, keepdims=True)
    var = jnp.mean(x * x, axis=-1, keepdims=True) - mean * mean
    inv = lax.rsqrt(var + EPS)
    o_ref[0] = (x - mean) * inv * g_ref[0][None, :] + b_ref[0][None, :]


_tc_ln = pl.pallas_call(
    _tc_ln_body,
    grid=(S // TBLK, B),
    in_specs=[
        pl.BlockSpec((1, TBLK, HIDDEN), lambda j, b: (b, j, 0)),
        pl.BlockSpec((TBLK, HIDDEN), lambda j, b: (j, 0)),
        pl.BlockSpec((1, 1, TBLK), lambda j, b: (b, 0, j)),
        pl.BlockSpec((1, HIDDEN), lambda j, b: (0, 0)),
        pl.BlockSpec((1, HIDDEN), lambda j, b: (0, 0)),
        pl.BlockSpec((1, HIDDEN), lambda j, b: (0, 0)),
    ],
    out_specs=pl.BlockSpec((1, TBLK, HIDDEN), lambda j, b: (b, j, 0)),
    out_shape=jax.ShapeDtypeStruct((B, S, HIDDEN), jnp.float32),
)


def kernel(input_ids, token_type_ids, word_table, pos_table, type_table,
           name_table, gamma, beta):
    ids = input_ids.astype(jnp.int32).reshape(N_TOK)
    gathered = _sc_gather(ids, word_table).reshape(B, S, HIDDEN)
    pos_plus = pos_table + name_table[0][None, :] + type_table[0][None, :]
    diff = (type_table[1] - type_table[0])[None, :]
    tf = token_type_ids.astype(jnp.float32).reshape(B, 1, S)
    out = _tc_ln(gathered, pos_plus, tf, diff, gamma[None, :], beta[None, :])
    return out


# speedup vs baseline: 2.7091x; 1.1719x over previous
import functools

import jax
import jax.numpy as jnp
from jax import lax
from jax.experimental import pallas as pl
from jax.experimental.pallas import tpu as pltpu
from jax.experimental.pallas import tpu_sc as plsc

B = 32
S = 512
HIDDEN = 768
EPS = 1e-12
N_TOK = B * S
CHUNK = 64
NCHUNK = S // CHUNK
TBLK = 512


def _sc_gather_body(ids_hbm, word_hbm, out_hbm, ids_v, buf0, buf1,
                    sem_g0, sem_g1, sem_o0, sem_o1):
    wid = lax.axis_index("s") * 2 + lax.axis_index("c")
    row_base = pl.multiple_of(wid * S, S)
    pltpu.sync_copy(ids_hbm.at[pl.ds(row_base, S)], ids_v)

    bufs = (buf0, buf1)
    gsems = (sem_g0, sem_g1)
    osems = (sem_o0, sem_o1)

    def start_gather(c):
        p = c % 2
        return pltpu.async_copy(
            word_hbm.at[ids_v.at[pl.ds(c * CHUNK, CHUNK)]], bufs[p], gsems[p])

    def start_out(c):
        p = c % 2
        return pltpu.async_copy(
            bufs[p], out_hbm.at[pl.ds(row_base + c * CHUNK, CHUNK)], osems[p])

    gather_h = {0: start_gather(0)}
    out_h = {}
    for c in range(NCHUNK):
        if c + 1 < NCHUNK:
            if c - 1 >= 0:
                out_h[c - 1].wait()
            gather_h[c + 1] = start_gather(c + 1)
        gather_h[c].wait()
        out_h[c] = start_out(c)
    out_h[NCHUNK - 2].wait()
    out_h[NCHUNK - 1].wait()


_sc_gather = functools.partial(
    pl.kernel,
    mesh=plsc.VectorSubcoreMesh(core_axis_name="c", subcore_axis_name="s"),
    out_type=jax.ShapeDtypeStruct((N_TOK, HIDDEN), jnp.float32),
    scratch_types=[
        pltpu.VMEM((S,), jnp.int32),
        pltpu.VMEM((CHUNK, HIDDEN), jnp.float32),
        pltpu.VMEM((CHUNK, HIDDEN), jnp.float32),
        pltpu.SemaphoreType.DMA,
        pltpu.SemaphoreType.DMA,
        pltpu.SemaphoreType.DMA,
        pltpu.SemaphoreType.DMA,
    ],
)(_sc_gather_body)


def _tc_ln_body(gath_ref, pp_ref, tf_ref, diff_ref, g_ref, b_ref, o_ref):
    x = (gath_ref[0] + pp_ref[...]
         + tf_ref[0, 0, :][:, None] * diff_ref[0][None, :])
    mean = jnp.mean(x, axis=-1, keepdims=True)
    var = jnp.mean(x * x, axis=-1, keepdims=True) - mean * mean
    inv = lax.rsqrt(var + EPS)
    o_ref[0] = (x - mean) * inv * g_ref[0][None, :] + b_ref[0][None, :]


_tc_ln = pl.pallas_call(
    _tc_ln_body,
    grid=(B,),
    in_specs=[
        pl.BlockSpec((1, TBLK, HIDDEN), lambda b: (b, 0, 0)),
        pl.BlockSpec((TBLK, HIDDEN), lambda b: (0, 0)),
        pl.BlockSpec((1, 1, TBLK), lambda b: (b, 0, 0)),
        pl.BlockSpec((1, HIDDEN), lambda b: (0, 0)),
        pl.BlockSpec((1, HIDDEN), lambda b: (0, 0)),
        pl.BlockSpec((1, HIDDEN), lambda b: (0, 0)),
    ],
    out_specs=pl.BlockSpec((1, TBLK, HIDDEN), lambda b: (b, 0, 0)),
    out_shape=jax.ShapeDtypeStruct((B, S, HIDDEN), jnp.float32),
)


def kernel(input_ids, token_type_ids, word_table, pos_table, type_table,
           name_table, gamma, beta):
    ids = input_ids.astype(jnp.int32).reshape(N_TOK)
    gathered = _sc_gather(ids, word_table).reshape(B, S, HIDDEN)
    pos_plus = pos_table + name_table[0][None, :] + type_table[0][None, :]
    diff = (type_table[1] - type_table[0])[None, :]
    tf = token_type_ids.astype(jnp.float32).reshape(B, 1, S)
    out = _tc_ln(gathered, pos_plus, tf, diff, gamma[None, :], beta[None, :])
    return out


# speedup vs baseline: 2.8183x; 1.0403x over previous
import functools

import jax
import jax.numpy as jnp
from jax import lax
from jax.experimental import pallas as pl
from jax.experimental.pallas import tpu as pltpu
from jax.experimental.pallas import tpu_sc as plsc

B = 32
S = 512
HIDDEN = 768
EPS = 1e-12
CHUNK = 64
NW = 32
BH = B // 2


def _make_sc_gather(n_rows):
    n_tok = n_rows * S
    per_worker = n_tok // NW
    nchunk = per_worker // CHUNK

    def body(ids_hbm, word_hbm, out_hbm, ids_v, buf0, buf1,
             sem_g0, sem_g1, sem_o0, sem_o1):
        wid = lax.axis_index("s") * 2 + lax.axis_index("c")
        row_base = pl.multiple_of(wid * per_worker, per_worker)
        pltpu.sync_copy(ids_hbm.at[pl.ds(row_base, per_worker)], ids_v)

        bufs = (buf0, buf1)
        gsems = (sem_g0, sem_g1)
        osems = (sem_o0, sem_o1)

        def start_gather(c):
            p = c % 2
            return pltpu.async_copy(
                word_hbm.at[ids_v.at[pl.ds(c * CHUNK, CHUNK)]],
                bufs[p], gsems[p])

        def start_out(c):
            p = c % 2
            return pltpu.async_copy(
                bufs[p], out_hbm.at[pl.ds(row_base + c * CHUNK, CHUNK)],
                osems[p])

        out_h = {}
        gather_h = {0: start_gather(0)}
        for c in range(nchunk):
            if c + 1 < nchunk:
                if c - 1 >= 0:
                    out_h[c - 1].wait()
                gather_h[c + 1] = start_gather(c + 1)
            gather_h[c].wait()
            out_h[c] = start_out(c)
        out_h[nchunk - 2].wait()
        out_h[nchunk - 1].wait()

    return functools.partial(
        pl.kernel,
        mesh=plsc.VectorSubcoreMesh(core_axis_name="c", subcore_axis_name="s"),
        out_type=jax.ShapeDtypeStruct((n_tok, HIDDEN), jnp.float32),
        scratch_types=[
            pltpu.VMEM((per_worker,), jnp.int32),
            pltpu.VMEM((CHUNK, HIDDEN), jnp.float32),
            pltpu.VMEM((CHUNK, HIDDEN), jnp.float32),
            pltpu.SemaphoreType.DMA,
            pltpu.SemaphoreType.DMA,
            pltpu.SemaphoreType.DMA,
            pltpu.SemaphoreType.DMA,
        ],
    )(body)


_sc_gather_half = _make_sc_gather(BH)


def _ln_block(gath_ref, pp_ref, tf_ref, diff_ref, g_ref, b_ref):
    x = (gath_ref[0] + pp_ref[...]
         + tf_ref[0, 0, :][:, None] * diff_ref[0][None, :])
    mean = jnp.mean(x, axis=-1, keepdims=True)
    var = jnp.mean(x * x, axis=-1, keepdims=True) - mean * mean
    inv = lax.rsqrt(var + EPS)
    return (x - mean) * inv * g_ref[0][None, :] + b_ref[0][None, :]


def _tc_ln_h1_body(gath_ref, pp_ref, tf_ref, diff_ref, g_ref, b_ref, o_ref):
    o_ref[0] = _ln_block(gath_ref, pp_ref, tf_ref, diff_ref, g_ref, b_ref)


def _tc_ln_h2_body(prev_ref, gath_ref, pp_ref, tf_ref, diff_ref, g_ref,
                   b_ref, o_ref):
    del prev_ref
    o_ref[0] = _ln_block(gath_ref, pp_ref, tf_ref, diff_ref, g_ref, b_ref)


_SMALL_SPECS = [
    pl.BlockSpec((S, HIDDEN), lambda b: (0, 0)),
    pl.BlockSpec((1, 1, S), lambda b: (b, 0, 0)),
    pl.BlockSpec((1, HIDDEN), lambda b: (0, 0)),
    pl.BlockSpec((1, HIDDEN), lambda b: (0, 0)),
    pl.BlockSpec((1, HIDDEN), lambda b: (0, 0)),
]

_tc_ln_h1 = pl.pallas_call(
    _tc_ln_h1_body,
    grid=(BH,),
    in_specs=[pl.BlockSpec((1, S, HIDDEN), lambda b: (b, 0, 0))]
    + _SMALL_SPECS,
    out_specs=pl.BlockSpec((1, S, HIDDEN), lambda b: (b, 0, 0)),
    out_shape=jax.ShapeDtypeStruct((B, S, HIDDEN), jnp.float32),
)

_tc_ln_h2 = pl.pallas_call(
    _tc_ln_h2_body,
    grid=(BH,),
    in_specs=[pl.BlockSpec(memory_space=pl.ANY),
              pl.BlockSpec((1, S, HIDDEN), lambda b: (b, 0, 0))]
    + _SMALL_SPECS,
    out_specs=pl.BlockSpec((1, S, HIDDEN), lambda b: (b + BH, 0, 0)),
    out_shape=jax.ShapeDtypeStruct((B, S, HIDDEN), jnp.float32),
    input_output_aliases={0: 0},
)


def kernel(input_ids, token_type_ids, word_table, pos_table, type_table,
           name_table, gamma, beta):
    ids = input_ids.astype(jnp.int32).reshape(B * S)
    pos_plus = pos_table + name_table[0][None, :] + type_table[0][None, :]
    diff = (type_table[1] - type_table[0])[None, :]
    tf = token_type_ids.astype(jnp.float32).reshape(B, 1, S)
    gamma2 = gamma[None, :]
    beta2 = beta[None, :]

    g1 = _sc_gather_half(ids[:BH * S], word_table).reshape(BH, S, HIDDEN)
    g2 = _sc_gather_half(ids[BH * S:], word_table).reshape(BH, S, HIDDEN)
    o_a = _tc_ln_h1(g1, pos_plus, tf[:BH], diff, gamma2, beta2)
    out = _tc_ln_h2(o_a, g2, pos_plus, tf[BH:], diff, gamma2, beta2)
    return out
